# Initial kernel scaffold; baseline (speedup 1.0000x reference)
import jax
import jax.numpy as jnp
from jax.experimental import pallas as pl


def kernel(x, edge_index, edge_attr, W_down, b_down, p_pool, W_bottle, b_bottle, W_up, b_up, W_final, b_final):
    raise NotImplementedError("write your pallas kernel here")


# speedup vs baseline: 2.6960x; 2.6960x over previous
import functools

import jax
import jax.numpy as jnp
from jax import lax
from jax.experimental import pallas as pl
from jax.experimental.pallas import tpu as pltpu
from jax.experimental.pallas import tpu_sc as plsc

N_NODES = 10000
N_PAD = 10240
E_EDGES = 320000
E_PAD = 327680
K_KEEP = 5000
K_PAD = 5120
NC, NS = 2, 16
NW = NC * NS
EPW = E_PAD // NW
CH = 128
NCHUNK = EPW // CH
RPS = N_PAD // NS

_sc_mesh = plsc.VectorSubcoreMesh(core_axis_name="c", subcore_axis_name="s")


def _make_sc_scatter_add(D):
    @functools.partial(
        pl.kernel,
        mesh=_sc_mesh,
        out_type=jax.ShapeDtypeStruct((NC, N_PAD, D), jnp.float32),
        scratch_types=[
            pltpu.VMEM((CH,), jnp.int32),
            pltpu.VMEM((CH,), jnp.int32),
            pltpu.VMEM((CH, D), jnp.float32),
            pltpu.VMEM_SHARED((N_PAD, D), jnp.float32),
            pltpu.SemaphoreType.DMA,
        ],
    )
    def sc_scatter_add(table_hbm, src_hbm, dst_hbm, zeros_hbm, out_hbm,
                       idx_s, idx_d, rows, acc, sem):
        c = lax.axis_index("c")
        s = lax.axis_index("s")
        w = c * NS + s
        pltpu.sync_copy(zeros_hbm.at[pl.ds(s * RPS, RPS)],
                        acc.at[pl.ds(s * RPS, RPS)])
        plsc.subcore_barrier()

        def step(g, carry):
            base = w * EPW + g * CH
            pltpu.sync_copy(src_hbm.at[pl.ds(base, CH)], idx_s)
            pltpu.sync_copy(dst_hbm.at[pl.ds(base, CH)], idx_d)
            pltpu.async_copy(table_hbm.at[idx_s], rows, sem).wait()
            pltpu.sync_copy(rows, acc.at[idx_d], add=True)
            return carry

        lax.fori_loop(0, NCHUNK, step, 0)
        plsc.subcore_barrier()
        pltpu.sync_copy(acc.at[pl.ds(s * RPS, RPS)],
                        out_hbm.at[c, pl.ds(s * RPS, RPS)])

    return sc_scatter_add


_sc_scatter_add_128 = _make_sc_scatter_add(128)

_GCH = 80


@functools.partial(
    pl.kernel,
    mesh=_sc_mesh,
    out_type=jax.ShapeDtypeStruct((K_PAD, 128), jnp.float32),
    scratch_types=[
        pltpu.VMEM((_GCH,), jnp.int32),
        pltpu.VMEM((_GCH, 128), jnp.float32),
        pltpu.SemaphoreType.DMA,
    ],
)
def _sc_gather_rows(table_hbm, perm_hbm, out_hbm, idx, rows, sem):
    c = lax.axis_index("c")
    s = lax.axis_index("s")
    w = c * NS + s

    def step(g, carry):
        base = w * (K_PAD // NW) + g * _GCH
        pltpu.sync_copy(perm_hbm.at[pl.ds(base, _GCH)], idx)
        pltpu.async_copy(table_hbm.at[idx], rows, sem).wait()
        pltpu.sync_copy(rows, out_hbm.at[pl.ds(base, _GCH)])
        return carry

    lax.fori_loop(0, K_PAD // NW // _GCH, step, 0)


_BR = 1024
_GRID = N_PAD // _BR


def _dinv2_block(mk, dp):
    deg = dp[0, :, 0:1] + dp[1, :, 0:1]
    return mk[:, 0:1] * lax.rsqrt(deg + 1.0)


def _tc_c_body(x1_ref, sc_ref, mk_ref, dp_ref, wb_ref, h2_ref, hsa_ref, hsb_ref):
    x1 = x1_ref[...]
    xp = mk_ref[:, 0:1] * sc_ref[:, 0:1] * x1
    h2 = jnp.dot(xp, wb_ref[...], preferred_element_type=jnp.float32)
    dinv2 = _dinv2_block(mk_ref[...], dp_ref[...])
    hs2 = dinv2 * h2
    h2_ref[...] = h2
    hsa_ref[...] = hs2[:, :128]
    hsb_ref[...] = hs2[:, 128:]


def _tc_d_body(a2a_ref, a2b_ref, h2_ref, x1_ref, mk_ref, dp_ref,
               wua_ref, wub_ref, bb_ref, h3_ref, hs3_ref):
    dinv2 = _dinv2_block(mk_ref[...], dp_ref[...])
    acc2a = a2a_ref[0] + a2a_ref[1]
    acc2b = a2b_ref[0] + a2b_ref[1]
    h2 = h2_ref[...]
    xb = jnp.concatenate(
        [dinv2 * acc2a + dinv2 * dinv2 * h2[:, :128],
         dinv2 * acc2b + dinv2 * dinv2 * h2[:, 128:]], axis=1) + bb_ref[...]
    h3 = (jnp.dot(xb, wua_ref[...], preferred_element_type=jnp.float32)
          + jnp.dot(x1_ref[...], wub_ref[...], preferred_element_type=jnp.float32))
    h3_ref[...] = h3
    hs3_ref[...] = dinv2 * h3


def _tc_e_body(a3_ref, h3_ref, mk_ref, dp_ref, wf_ref, bu_ref, h4_ref, hs4_ref):
    dinv2 = _dinv2_block(mk_ref[...], dp_ref[...])
    acc3 = a3_ref[0] + a3_ref[1]
    xu = dinv2 * acc3 + dinv2 * dinv2 * h3_ref[...] + bu_ref[...]
    h4 = jnp.dot(xu, wf_ref[...], preferred_element_type=jnp.float32)
    h4_ref[...] = h4
    hs4_ref[...] = dinv2 * h4


def _tc_f_body(a4_ref, h4_ref, mk_ref, dp_ref, bf_ref, out_ref):
    dinv2 = _dinv2_block(mk_ref[...], dp_ref[...])
    acc4 = a4_ref[0] + a4_ref[1]
    out_ref[...] = dinv2 * acc4 + dinv2 * dinv2 * h4_ref[...] + bf_ref[...]


def _rows_spec(width):
    return pl.BlockSpec((_BR, width), lambda i: (i, 0))


def _part_spec(width):
    return pl.BlockSpec((2, _BR, width), lambda i: (0, i, 0))


def _full_spec(shape):
    return pl.BlockSpec(shape, lambda i: tuple(0 for _ in shape))


def _tc_call(body, in_specs, out_specs, out_shapes):
    return pl.pallas_call(
        body,
        grid=(_GRID,),
        in_specs=in_specs,
        out_specs=out_specs,
        out_shape=out_shapes,
    )


def kernel(x, edge_index, edge_attr, W_down, b_down, p_pool,
           W_bottle, b_bottle, W_up, b_up, W_final, b_final):
    N = x.shape[0]
    src = edge_index[0]
    dst = edge_index[1]
    E = src.shape[0]

    h1 = x @ W_down
    deg1 = jnp.zeros((N,), jnp.float32).at[dst].add(jnp.ones((E,), jnp.float32)) + 1.0
    dinv1 = 1.0 / jnp.sqrt(deg1)
    norm1 = dinv1[src] * dinv1[dst]
    x1 = jnp.zeros((N, 128), jnp.float32).at[dst].add(norm1[:, None] * h1[src])
    x1 = x1 + (dinv1 * dinv1)[:, None] * h1
    x1 = x1 + b_down
    score = jnp.tanh((x1 @ p_pool) / jnp.sqrt(jnp.sum(p_pool * p_pool)))
    k = (N + 1) // 2
    topscore, perm = jax.lax.top_k(score, k)
    mask = jnp.zeros((N,), jnp.float32).at[perm].set(1.0)

    pad_e = E_PAD - E
    src_p = jnp.concatenate([src, jnp.full((pad_e,), N_NODES, jnp.int32)])
    dst_p = jnp.concatenate([dst, jnp.full((pad_e,), N_NODES, jnp.int32)])
    perm_p = jnp.concatenate(
        [perm.astype(jnp.int32), jnp.full((K_PAD - k,), N_NODES, jnp.int32)])
    zeros128 = jnp.zeros((N_PAD, 128), jnp.float32)
    x1p = jnp.pad(x1, ((0, N_PAD - N), (0, 0)))
    mask_b = jnp.pad(jnp.broadcast_to(mask[:, None], (N, 128)),
                     ((0, N_PAD - N), (0, 0)))
    score_b = jnp.pad(jnp.broadcast_to(score[:, None], (N, 128)),
                      ((0, N_PAD - N), (0, 0)))
    bb = b_bottle.reshape(1, 256)
    bu = b_up.reshape(1, 128)
    bf = b_final.reshape(1, 128)
    wua = W_up[:256]
    wub = W_up[256:]

    deg_p = _sc_scatter_add_128(mask_b, src_p, dst_p, zeros128)

    h2, hs2a, hs2b = _tc_call(
        _tc_c_body,
        [_rows_spec(128), _rows_spec(128), _rows_spec(128), _part_spec(128),
         _full_spec((128, 256))],
        [_rows_spec(256), _rows_spec(128), _rows_spec(128)],
        [jax.ShapeDtypeStruct((N_PAD, 256), jnp.float32),
         jax.ShapeDtypeStruct((N_PAD, 128), jnp.float32),
         jax.ShapeDtypeStruct((N_PAD, 128), jnp.float32)],
    )(x1p, score_b, mask_b, deg_p, W_bottle)

    a2a = _sc_scatter_add_128(hs2a, src_p, dst_p, zeros128)
    a2b = _sc_scatter_add_128(hs2b, src_p, dst_p, zeros128)

    h3, hs3 = _tc_call(
        _tc_d_body,
        [_part_spec(128), _part_spec(128), _rows_spec(256), _rows_spec(128),
         _rows_spec(128), _part_spec(128), _full_spec((256, 128)),
         _full_spec((128, 128)), _full_spec((1, 256))],
        [_rows_spec(128), _rows_spec(128)],
        [jax.ShapeDtypeStruct((N_PAD, 128), jnp.float32),
         jax.ShapeDtypeStruct((N_PAD, 128), jnp.float32)],
    )(a2a, a2b, h2, x1p, mask_b, deg_p, wua, wub, bb)

    a3 = _sc_scatter_add_128(hs3, src_p, dst_p, zeros128)

    h4, hs4 = _tc_call(
        _tc_e_body,
        [_part_spec(128), _rows_spec(128), _rows_spec(128), _part_spec(128),
         _full_spec((128, 128)), _full_spec((1, 128))],
        [_rows_spec(128), _rows_spec(128)],
        [jax.ShapeDtypeStruct((N_PAD, 128), jnp.float32),
         jax.ShapeDtypeStruct((N_PAD, 128), jnp.float32)],
    )(a3, h3, mask_b, deg_p, W_final, bu)

    a4 = _sc_scatter_add_128(hs4, src_p, dst_p, zeros128)

    out_full = _tc_call(
        _tc_f_body,
        [_part_spec(128), _rows_spec(128), _rows_spec(128), _part_spec(128),
         _full_spec((1, 128))],
        _rows_spec(128),
        jax.ShapeDtypeStruct((N_PAD, 128), jnp.float32),
    )(a4, h4, mask_b, deg_p, bf)

    out_p = _sc_gather_rows(out_full, perm_p)
    return out_p[:k]


# speedup vs baseline: 2.8934x; 1.0732x over previous
import functools

import jax
import jax.numpy as jnp
from jax import lax
from jax.experimental import pallas as pl
from jax.experimental.pallas import tpu as pltpu
from jax.experimental.pallas import tpu_sc as plsc

N_NODES = 10000
N_PAD = 10240
E_EDGES = 320000
E_PAD = 327680
K_KEEP = 5000
K_PAD = 5120
NC, NS = 2, 16
NW = NC * NS
EPW = E_PAD // NW
CH = 128
NCHUNK = EPW // CH
RPS = N_PAD // NS

_sc_mesh = plsc.VectorSubcoreMesh(core_axis_name="c", subcore_axis_name="s")


def _make_sc_scatter_add(D):
    @functools.partial(
        pl.kernel,
        mesh=_sc_mesh,
        out_type=jax.ShapeDtypeStruct((NC, N_PAD, D), jnp.float32),
        scratch_types=[
            pltpu.VMEM((EPW,), jnp.int32),
            pltpu.VMEM((CH,), jnp.int32),
            pltpu.VMEM((CH,), jnp.int32),
            pltpu.VMEM((CH, D), jnp.float32),
            pltpu.VMEM((CH, D), jnp.float32),
            pltpu.VMEM_SHARED((N_PAD, D), jnp.float32),
            pltpu.SemaphoreType.DMA,
            pltpu.SemaphoreType.DMA,
            pltpu.SemaphoreType.DMA,
            pltpu.SemaphoreType.DMA,
        ],
    )
    def sc_scatter_add(table_hbm, src_hbm, dst_hbm, zeros_hbm, out_hbm,
                       idx_s, di0, di1, rows0, rows1, acc,
                       sem0, sem1, semd0, semd1):
        c = lax.axis_index("c")
        s = lax.axis_index("s")
        w = c * NS + s
        base = w * EPW
        pltpu.sync_copy(src_hbm.at[pl.ds(base, EPW)], idx_s)
        pltpu.sync_copy(zeros_hbm.at[pl.ds(s * RPS, RPS)],
                        acc.at[pl.ds(s * RPS, RPS)])
        plsc.subcore_barrier()

        def wait_rows(buf, sem):
            pltpu.make_async_copy(table_hbm.at[idx_s.at[pl.ds(0, CH)]],
                                  buf, sem).wait()

        def wait_idx(buf, sem):
            pltpu.make_async_copy(dst_hbm.at[pl.ds(0, CH)], buf, sem).wait()

        pltpu.async_copy(dst_hbm.at[pl.ds(base, CH)], di0, semd0)
        pltpu.async_copy(table_hbm.at[idx_s.at[pl.ds(0, CH)]], rows0, sem0)

        def step(t, carry):
            g0 = t * 2
            pltpu.async_copy(dst_hbm.at[pl.ds(base + (g0 + 1) * CH, CH)],
                             di1, semd1)
            pltpu.async_copy(table_hbm.at[idx_s.at[pl.ds((g0 + 1) * CH, CH)]],
                             rows1, sem1)
            wait_rows(rows0, sem0)
            wait_idx(di0, semd0)
            pltpu.sync_copy(rows0, acc.at[di0], add=True)

            @pl.when(g0 + 2 < NCHUNK)
            def _():
                pltpu.async_copy(dst_hbm.at[pl.ds(base + (g0 + 2) * CH, CH)],
                                 di0, semd0)
                pltpu.async_copy(
                    table_hbm.at[idx_s.at[pl.ds((g0 + 2) * CH, CH)]], rows0, sem0)

            wait_rows(rows1, sem1)
            wait_idx(di1, semd1)
            pltpu.sync_copy(rows1, acc.at[di1], add=True)
            return carry

        lax.fori_loop(0, NCHUNK // 2, step, 0)
        plsc.subcore_barrier()
        pltpu.sync_copy(acc.at[pl.ds(s * RPS, RPS)],
                        out_hbm.at[c, pl.ds(s * RPS, RPS)])

    return sc_scatter_add


_sc_scatter_add_128 = _make_sc_scatter_add(128)

_GCH = 80


@functools.partial(
    pl.kernel,
    mesh=_sc_mesh,
    out_type=jax.ShapeDtypeStruct((K_PAD, 128), jnp.float32),
    scratch_types=[
        pltpu.VMEM((_GCH,), jnp.int32),
        pltpu.VMEM((_GCH, 128), jnp.float32),
        pltpu.SemaphoreType.DMA,
    ],
)
def _sc_gather_rows(table_hbm, perm_hbm, out_hbm, idx, rows, sem):
    c = lax.axis_index("c")
    s = lax.axis_index("s")
    w = c * NS + s

    def step(g, carry):
        base = w * (K_PAD // NW) + g * _GCH
        pltpu.sync_copy(perm_hbm.at[pl.ds(base, _GCH)], idx)
        pltpu.async_copy(table_hbm.at[idx], rows, sem).wait()
        pltpu.sync_copy(rows, out_hbm.at[pl.ds(base, _GCH)])
        return carry

    lax.fori_loop(0, K_PAD // NW // _GCH, step, 0)


_BR = 1024
_GRID = N_PAD // _BR


def _dinv2_block(mk, dp):
    deg = dp[0, :, 0:1] + dp[1, :, 0:1]
    return mk[:, 0:1] * lax.rsqrt(deg + 1.0)


def _tc_c_body(x1_ref, sc_ref, mk_ref, dp_ref, wb_ref, h2_ref, hsa_ref, hsb_ref):
    x1 = x1_ref[...]
    xp = mk_ref[:, 0:1] * sc_ref[:, 0:1] * x1
    h2 = jnp.dot(xp, wb_ref[...], preferred_element_type=jnp.float32)
    dinv2 = _dinv2_block(mk_ref[...], dp_ref[...])
    hs2 = dinv2 * h2
    h2_ref[...] = h2
    hsa_ref[...] = hs2[:, :128]
    hsb_ref[...] = hs2[:, 128:]


def _tc_d_body(a2a_ref, a2b_ref, h2_ref, x1_ref, mk_ref, dp_ref,
               wua_ref, wub_ref, bb_ref, h3_ref, hs3_ref):
    dinv2 = _dinv2_block(mk_ref[...], dp_ref[...])
    acc2a = a2a_ref[0] + a2a_ref[1]
    acc2b = a2b_ref[0] + a2b_ref[1]
    h2 = h2_ref[...]
    xb = jnp.concatenate(
        [dinv2 * acc2a + dinv2 * dinv2 * h2[:, :128],
         dinv2 * acc2b + dinv2 * dinv2 * h2[:, 128:]], axis=1) + bb_ref[...]
    h3 = (jnp.dot(xb, wua_ref[...], preferred_element_type=jnp.float32)
          + jnp.dot(x1_ref[...], wub_ref[...], preferred_element_type=jnp.float32))
    h3_ref[...] = h3
    hs3_ref[...] = dinv2 * h3


def _tc_e_body(a3_ref, h3_ref, mk_ref, dp_ref, wf_ref, bu_ref, h4_ref, hs4_ref):
    dinv2 = _dinv2_block(mk_ref[...], dp_ref[...])
    acc3 = a3_ref[0] + a3_ref[1]
    xu = dinv2 * acc3 + dinv2 * dinv2 * h3_ref[...] + bu_ref[...]
    h4 = jnp.dot(xu, wf_ref[...], preferred_element_type=jnp.float32)
    h4_ref[...] = h4
    hs4_ref[...] = dinv2 * h4


def _tc_f_body(a4_ref, h4_ref, mk_ref, dp_ref, bf_ref, out_ref):
    dinv2 = _dinv2_block(mk_ref[...], dp_ref[...])
    acc4 = a4_ref[0] + a4_ref[1]
    out_ref[...] = dinv2 * acc4 + dinv2 * dinv2 * h4_ref[...] + bf_ref[...]


def _rows_spec(width):
    return pl.BlockSpec((_BR, width), lambda i: (i, 0))


def _part_spec(width):
    return pl.BlockSpec((2, _BR, width), lambda i: (0, i, 0))


def _full_spec(shape):
    return pl.BlockSpec(shape, lambda i: tuple(0 for _ in shape))


def _tc_call(body, in_specs, out_specs, out_shapes):
    return pl.pallas_call(
        body,
        grid=(_GRID,),
        in_specs=in_specs,
        out_specs=out_specs,
        out_shape=out_shapes,
    )


def kernel(x, edge_index, edge_attr, W_down, b_down, p_pool,
           W_bottle, b_bottle, W_up, b_up, W_final, b_final):
    N = x.shape[0]
    src = edge_index[0]
    dst = edge_index[1]
    E = src.shape[0]

    h1 = x @ W_down
    deg1 = jnp.zeros((N,), jnp.float32).at[dst].add(jnp.ones((E,), jnp.float32)) + 1.0
    dinv1 = 1.0 / jnp.sqrt(deg1)
    norm1 = dinv1[src] * dinv1[dst]
    x1 = jnp.zeros((N, 128), jnp.float32).at[dst].add(norm1[:, None] * h1[src])
    x1 = x1 + (dinv1 * dinv1)[:, None] * h1
    x1 = x1 + b_down
    score = jnp.tanh((x1 @ p_pool) / jnp.sqrt(jnp.sum(p_pool * p_pool)))
    k = (N + 1) // 2
    topscore, perm = jax.lax.top_k(score, k)
    mask = jnp.zeros((N,), jnp.float32).at[perm].set(1.0)

    pad_e = E_PAD - E
    src_p = jnp.concatenate([src, jnp.full((pad_e,), N_NODES, jnp.int32)])
    dst_p = jnp.concatenate([dst, jnp.full((pad_e,), N_NODES, jnp.int32)])
    perm_p = jnp.concatenate(
        [perm.astype(jnp.int32), jnp.full((K_PAD - k,), N_NODES, jnp.int32)])
    zeros128 = jnp.zeros((N_PAD, 128), jnp.float32)
    x1p = jnp.pad(x1, ((0, N_PAD - N), (0, 0)))
    mask_b = jnp.pad(jnp.broadcast_to(mask[:, None], (N, 128)),
                     ((0, N_PAD - N), (0, 0)))
    score_b = jnp.pad(jnp.broadcast_to(score[:, None], (N, 128)),
                      ((0, N_PAD - N), (0, 0)))
    bb = b_bottle.reshape(1, 256)
    bu = b_up.reshape(1, 128)
    bf = b_final.reshape(1, 128)
    wua = W_up[:256]
    wub = W_up[256:]

    deg_p = _sc_scatter_add_128(mask_b, src_p, dst_p, zeros128)

    h2, hs2a, hs2b = _tc_call(
        _tc_c_body,
        [_rows_spec(128), _rows_spec(128), _rows_spec(128), _part_spec(128),
         _full_spec((128, 256))],
        [_rows_spec(256), _rows_spec(128), _rows_spec(128)],
        [jax.ShapeDtypeStruct((N_PAD, 256), jnp.float32),
         jax.ShapeDtypeStruct((N_PAD, 128), jnp.float32),
         jax.ShapeDtypeStruct((N_PAD, 128), jnp.float32)],
    )(x1p, score_b, mask_b, deg_p, W_bottle)

    a2a = _sc_scatter_add_128(hs2a, src_p, dst_p, zeros128)
    a2b = _sc_scatter_add_128(hs2b, src_p, dst_p, zeros128)

    h3, hs3 = _tc_call(
        _tc_d_body,
        [_part_spec(128), _part_spec(128), _rows_spec(256), _rows_spec(128),
         _rows_spec(128), _part_spec(128), _full_spec((256, 128)),
         _full_spec((128, 128)), _full_spec((1, 256))],
        [_rows_spec(128), _rows_spec(128)],
        [jax.ShapeDtypeStruct((N_PAD, 128), jnp.float32),
         jax.ShapeDtypeStruct((N_PAD, 128), jnp.float32)],
    )(a2a, a2b, h2, x1p, mask_b, deg_p, wua, wub, bb)

    a3 = _sc_scatter_add_128(hs3, src_p, dst_p, zeros128)

    h4, hs4 = _tc_call(
        _tc_e_body,
        [_part_spec(128), _rows_spec(128), _rows_spec(128), _part_spec(128),
         _full_spec((128, 128)), _full_spec((1, 128))],
        [_rows_spec(128), _rows_spec(128)],
        [jax.ShapeDtypeStruct((N_PAD, 128), jnp.float32),
         jax.ShapeDtypeStruct((N_PAD, 128), jnp.float32)],
    )(a3, h3, mask_b, deg_p, W_final, bu)

    a4 = _sc_scatter_add_128(hs4, src_p, dst_p, zeros128)

    out_full = _tc_call(
        _tc_f_body,
        [_part_spec(128), _rows_spec(128), _rows_spec(128), _part_spec(128),
         _full_spec((1, 128))],
        _rows_spec(128),
        jax.ShapeDtypeStruct((N_PAD, 128), jnp.float32),
    )(a4, h4, mask_b, deg_p, bf)

    out_p = _sc_gather_rows(out_full, perm_p)
    return out_p[:k]


# speedup vs baseline: 3.1203x; 1.0784x over previous
import functools

import jax
import jax.numpy as jnp
from jax import lax
from jax.experimental import pallas as pl
from jax.experimental.pallas import tpu as pltpu
from jax.experimental.pallas import tpu_sc as plsc

N_NODES = 10000
N_PAD = 10240
E_EDGES = 320000
E_PAD = 327680
K_KEEP = 5000
K_PAD = 5120
NC, NS = 2, 16
NW = NC * NS
EPW = E_PAD // NW
CH = 128
NCHUNK = EPW // CH
RPS = N_PAD // NS

_sc_mesh = plsc.VectorSubcoreMesh(core_axis_name="c", subcore_axis_name="s")


def _make_sc_scatter_add(D):
    @functools.partial(
        pl.kernel,
        mesh=_sc_mesh,
        out_type=jax.ShapeDtypeStruct((NC, N_PAD, D), jnp.float32),
        scratch_types=[
            pltpu.VMEM((EPW,), jnp.int32),
            pltpu.VMEM((CH,), jnp.int32),
            pltpu.VMEM((CH,), jnp.int32),
            pltpu.VMEM((CH, D), jnp.float32),
            pltpu.VMEM((CH, D), jnp.float32),
            pltpu.VMEM_SHARED((N_PAD, D), jnp.float32),
            pltpu.SemaphoreType.DMA,
            pltpu.SemaphoreType.DMA,
            pltpu.SemaphoreType.DMA,
            pltpu.SemaphoreType.DMA,
        ],
    )
    def sc_scatter_add(table_hbm, src_hbm, dst_hbm, zeros_hbm, out_hbm,
                       idx_s, di0, di1, rows0, rows1, acc,
                       sem0, sem1, semd0, semd1):
        c = lax.axis_index("c")
        s = lax.axis_index("s")
        w = c * NS + s
        base = w * EPW
        pltpu.sync_copy(src_hbm.at[pl.ds(base, EPW)], idx_s)
        pltpu.sync_copy(zeros_hbm.at[pl.ds(s * RPS, RPS)],
                        acc.at[pl.ds(s * RPS, RPS)])
        plsc.subcore_barrier()

        def wait_rows(buf, sem):
            pltpu.make_async_copy(table_hbm.at[idx_s.at[pl.ds(0, CH)]],
                                  buf, sem).wait()

        def wait_idx(buf, sem):
            pltpu.make_async_copy(dst_hbm.at[pl.ds(0, CH)], buf, sem).wait()

        pltpu.async_copy(dst_hbm.at[pl.ds(base, CH)], di0, semd0)
        pltpu.async_copy(table_hbm.at[idx_s.at[pl.ds(0, CH)]], rows0, sem0)

        def step(t, carry):
            g0 = t * 2
            pltpu.async_copy(dst_hbm.at[pl.ds(base + (g0 + 1) * CH, CH)],
                             di1, semd1)
            pltpu.async_copy(table_hbm.at[idx_s.at[pl.ds((g0 + 1) * CH, CH)]],
                             rows1, sem1)
            wait_rows(rows0, sem0)
            wait_idx(di0, semd0)
            pltpu.sync_copy(rows0, acc.at[di0], add=True)

            @pl.when(g0 + 2 < NCHUNK)
            def _():
                pltpu.async_copy(dst_hbm.at[pl.ds(base + (g0 + 2) * CH, CH)],
                                 di0, semd0)
                pltpu.async_copy(
                    table_hbm.at[idx_s.at[pl.ds((g0 + 2) * CH, CH)]], rows0, sem0)

            wait_rows(rows1, sem1)
            wait_idx(di1, semd1)
            pltpu.sync_copy(rows1, acc.at[di1], add=True)
            return carry

        lax.fori_loop(0, NCHUNK // 2, step, 0)
        plsc.subcore_barrier()
        pltpu.sync_copy(acc.at[pl.ds(s * RPS, RPS)],
                        out_hbm.at[c, pl.ds(s * RPS, RPS)])

    return sc_scatter_add


_sc_scatter_add_128 = _make_sc_scatter_add(128)

@functools.partial(
    pl.kernel,
    mesh=_sc_mesh,
    out_type=jax.ShapeDtypeStruct((E_PAD, 128), jnp.float32),
    scratch_types=[
        pltpu.VMEM((EPW,), jnp.int32),
        pltpu.VMEM((CH, 128), jnp.float32),
        pltpu.VMEM((CH, 128), jnp.float32),
        pltpu.SemaphoreType.DMA,
        pltpu.SemaphoreType.DMA,
    ],
)
def _sc_gather_edges(table_hbm, src_hbm, out_hbm, idx_s, rows0, rows1,
                     sem0, sem1):
    c = lax.axis_index("c")
    s = lax.axis_index("s")
    w = c * NS + s
    base = w * EPW
    pltpu.sync_copy(src_hbm.at[pl.ds(base, EPW)], idx_s)

    def wait_rows(buf, sem):
        pltpu.make_async_copy(table_hbm.at[idx_s.at[pl.ds(0, CH)]],
                              buf, sem).wait()

    pltpu.async_copy(table_hbm.at[idx_s.at[pl.ds(0, CH)]], rows0, sem0)

    def step(t, carry):
        g0 = t * 2
        pltpu.async_copy(table_hbm.at[idx_s.at[pl.ds((g0 + 1) * CH, CH)]],
                         rows1, sem1)
        wait_rows(rows0, sem0)
        pltpu.sync_copy(rows0, out_hbm.at[pl.ds(base + g0 * CH, CH)])

        @pl.when(g0 + 2 < NCHUNK)
        def _():
            pltpu.async_copy(
                table_hbm.at[idx_s.at[pl.ds((g0 + 2) * CH, CH)]], rows0, sem0)

        wait_rows(rows1, sem1)
        pltpu.sync_copy(rows1, out_hbm.at[pl.ds(base + (g0 + 1) * CH, CH)])
        return carry

    lax.fori_loop(0, NCHUNK // 2, step, 0)


_GCH = 80


@functools.partial(
    pl.kernel,
    mesh=_sc_mesh,
    out_type=jax.ShapeDtypeStruct((K_PAD, 128), jnp.float32),
    scratch_types=[
        pltpu.VMEM((_GCH,), jnp.int32),
        pltpu.VMEM((_GCH, 128), jnp.float32),
        pltpu.SemaphoreType.DMA,
    ],
)
def _sc_gather_rows(table_hbm, perm_hbm, out_hbm, idx, rows, sem):
    c = lax.axis_index("c")
    s = lax.axis_index("s")
    w = c * NS + s

    def step(g, carry):
        base = w * (K_PAD // NW) + g * _GCH
        pltpu.sync_copy(perm_hbm.at[pl.ds(base, _GCH)], idx)
        pltpu.async_copy(table_hbm.at[idx], rows, sem).wait()
        pltpu.sync_copy(rows, out_hbm.at[pl.ds(base, _GCH)])
        return carry

    lax.fori_loop(0, K_PAD // NW // _GCH, step, 0)


_BR = 1024
_GRID = N_PAD // _BR


def _dinv2_block(mk, dp):
    deg = dp[0, :, 0:1] + dp[1, :, 0:1]
    return mk[:, 0:1] * lax.rsqrt(deg + 1.0)


def _tc_c_body(x1_ref, sc_ref, mk_ref, dp_ref, wb_ref, h2_ref, hsa_ref, hsb_ref):
    x1 = x1_ref[...]
    xp = mk_ref[:, 0:1] * sc_ref[:, 0:1] * x1
    h2 = jnp.dot(xp, wb_ref[...], preferred_element_type=jnp.float32)
    dinv2 = _dinv2_block(mk_ref[...], dp_ref[...])
    hs2 = dinv2 * h2
    h2_ref[...] = h2
    hsa_ref[...] = hs2[:, :128]
    hsb_ref[...] = hs2[:, 128:]


def _tc_d_body(a2a_ref, a2b_ref, h2_ref, x1_ref, mk_ref, dp_ref,
               wua_ref, wub_ref, bb_ref, h3_ref, hs3_ref):
    dinv2 = _dinv2_block(mk_ref[...], dp_ref[...])
    acc2a = a2a_ref[0] + a2a_ref[1]
    acc2b = a2b_ref[0] + a2b_ref[1]
    h2 = h2_ref[...]
    xb = jnp.concatenate(
        [dinv2 * acc2a + dinv2 * dinv2 * h2[:, :128],
         dinv2 * acc2b + dinv2 * dinv2 * h2[:, 128:]], axis=1) + bb_ref[...]
    h3 = (jnp.dot(xb, wua_ref[...], preferred_element_type=jnp.float32)
          + jnp.dot(x1_ref[...], wub_ref[...], preferred_element_type=jnp.float32))
    h3_ref[...] = h3
    hs3_ref[...] = dinv2 * h3


def _tc_e_body(a3_ref, h3_ref, mk_ref, dp_ref, wf_ref, bu_ref, h4_ref, hs4_ref):
    dinv2 = _dinv2_block(mk_ref[...], dp_ref[...])
    acc3 = a3_ref[0] + a3_ref[1]
    xu = dinv2 * acc3 + dinv2 * dinv2 * h3_ref[...] + bu_ref[...]
    h4 = jnp.dot(xu, wf_ref[...], preferred_element_type=jnp.float32)
    h4_ref[...] = h4
    hs4_ref[...] = dinv2 * h4


def _tc_f_body(a4_ref, h4_ref, mk_ref, dp_ref, bf_ref, out_ref):
    dinv2 = _dinv2_block(mk_ref[...], dp_ref[...])
    acc4 = a4_ref[0] + a4_ref[1]
    out_ref[...] = dinv2 * acc4 + dinv2 * dinv2 * h4_ref[...] + bf_ref[...]


def _rows_spec(width):
    return pl.BlockSpec((_BR, width), lambda i: (i, 0))


def _part_spec(width):
    return pl.BlockSpec((2, _BR, width), lambda i: (0, i, 0))


def _full_spec(shape):
    return pl.BlockSpec(shape, lambda i: tuple(0 for _ in shape))


def _tc_call(body, in_specs, out_specs, out_shapes):
    return pl.pallas_call(
        body,
        grid=(_GRID,),
        in_specs=in_specs,
        out_specs=out_specs,
        out_shape=out_shapes,
    )


def kernel(x, edge_index, edge_attr, W_down, b_down, p_pool,
           W_bottle, b_bottle, W_up, b_up, W_final, b_final):
    N = x.shape[0]
    src = edge_index[0]
    dst = edge_index[1]
    E = src.shape[0]

    pad_e = E_PAD - E
    src_p = jnp.concatenate([src, jnp.full((pad_e,), N_NODES, jnp.int32)])
    dst_p = jnp.concatenate([dst, jnp.full((pad_e,), N_NODES, jnp.int32)])

    h1 = x @ W_down
    deg1 = jnp.zeros((N,), jnp.float32).at[dst].add(jnp.ones((E,), jnp.float32)) + 1.0
    dinv1 = 1.0 / jnp.sqrt(deg1)
    norm1 = dinv1[src] * dinv1[dst]
    g1 = _sc_gather_edges(jnp.pad(h1, ((0, N_PAD - N), (0, 0))), src_p)[:E]
    x1 = jnp.zeros((N, 128), jnp.float32).at[dst].add(norm1[:, None] * g1)
    x1 = x1 + (dinv1 * dinv1)[:, None] * h1
    x1 = x1 + b_down
    score = jnp.tanh((x1 @ p_pool) / jnp.sqrt(jnp.sum(p_pool * p_pool)))
    k = (N + 1) // 2
    topscore, perm = jax.lax.top_k(score, k)
    mask = jnp.zeros((N,), jnp.float32).at[perm].set(1.0)
    perm_p = jnp.concatenate(
        [perm.astype(jnp.int32), jnp.full((K_PAD - k,), N_NODES, jnp.int32)])
    zeros128 = jnp.zeros((N_PAD, 128), jnp.float32)
    x1p = jnp.pad(x1, ((0, N_PAD - N), (0, 0)))
    mask_b = jnp.pad(jnp.broadcast_to(mask[:, None], (N, 128)),
                     ((0, N_PAD - N), (0, 0)))
    score_b = jnp.pad(jnp.broadcast_to(score[:, None], (N, 128)),
                      ((0, N_PAD - N), (0, 0)))
    bb = b_bottle.reshape(1, 256)
    bu = b_up.reshape(1, 128)
    bf = b_final.reshape(1, 128)
    wua = W_up[:256]
    wub = W_up[256:]

    deg_p = _sc_scatter_add_128(mask_b, src_p, dst_p, zeros128)

    h2, hs2a, hs2b = _tc_call(
        _tc_c_body,
        [_rows_spec(128), _rows_spec(128), _rows_spec(128), _part_spec(128),
         _full_spec((128, 256))],
        [_rows_spec(256), _rows_spec(128), _rows_spec(128)],
        [jax.ShapeDtypeStruct((N_PAD, 256), jnp.float32),
         jax.ShapeDtypeStruct((N_PAD, 128), jnp.float32),
         jax.ShapeDtypeStruct((N_PAD, 128), jnp.float32)],
    )(x1p, score_b, mask_b, deg_p, W_bottle)

    a2a = _sc_scatter_add_128(hs2a, src_p, dst_p, zeros128)
    a2b = _sc_scatter_add_128(hs2b, src_p, dst_p, zeros128)

    h3, hs3 = _tc_call(
        _tc_d_body,
        [_part_spec(128), _part_spec(128), _rows_spec(256), _rows_spec(128),
         _rows_spec(128), _part_spec(128), _full_spec((256, 128)),
         _full_spec((128, 128)), _full_spec((1, 256))],
        [_rows_spec(128), _rows_spec(128)],
        [jax.ShapeDtypeStruct((N_PAD, 128), jnp.float32),
         jax.ShapeDtypeStruct((N_PAD, 128), jnp.float32)],
    )(a2a, a2b, h2, x1p, mask_b, deg_p, wua, wub, bb)

    a3 = _sc_scatter_add_128(hs3, src_p, dst_p, zeros128)

    h4, hs4 = _tc_call(
        _tc_e_body,
        [_part_spec(128), _rows_spec(128), _rows_spec(128), _part_spec(128),
         _full_spec((128, 128)), _full_spec((1, 128))],
        [_rows_spec(128), _rows_spec(128)],
        [jax.ShapeDtypeStruct((N_PAD, 128), jnp.float32),
         jax.ShapeDtypeStruct((N_PAD, 128), jnp.float32)],
    )(a3, h3, mask_b, deg_p, W_final, bu)

    a4 = _sc_scatter_add_128(hs4, src_p, dst_p, zeros128)

    out_full = _tc_call(
        _tc_f_body,
        [_part_spec(128), _rows_spec(128), _rows_spec(128), _part_spec(128),
         _full_spec((1, 128))],
        _rows_spec(128),
        jax.ShapeDtypeStruct((N_PAD, 128), jnp.float32),
    )(a4, h4, mask_b, deg_p, bf)

    out_p = _sc_gather_rows(out_full, perm_p)
    return out_p[:k]


# speedup vs baseline: 3.1207x; 1.0001x over previous
import functools

import jax
import jax.numpy as jnp
from jax import lax
from jax.experimental import pallas as pl
from jax.experimental.pallas import tpu as pltpu
from jax.experimental.pallas import tpu_sc as plsc

N_NODES = 10000
N_PAD = 10240
E_EDGES = 320000
E_PAD = 327680
K_KEEP = 5000
K_PAD = 5120
NC, NS = 2, 16
NW = NC * NS
EPW = E_PAD // NW
CH = 128
NCHUNK = EPW // CH
RPS = N_PAD // NS

_sc_mesh = plsc.VectorSubcoreMesh(core_axis_name="c", subcore_axis_name="s")


def _make_sc_scatter_add(D):
    @functools.partial(
        pl.kernel,
        mesh=_sc_mesh,
        out_type=jax.ShapeDtypeStruct((NC, N_PAD, D), jnp.float32),
        scratch_types=[
            pltpu.VMEM((EPW,), jnp.int32),
            pltpu.VMEM((CH,), jnp.int32),
            pltpu.VMEM((CH,), jnp.int32),
            pltpu.VMEM((CH, D), jnp.float32),
            pltpu.VMEM((CH, D), jnp.float32),
            pltpu.VMEM_SHARED((N_PAD, D), jnp.float32),
            pltpu.SemaphoreType.DMA,
            pltpu.SemaphoreType.DMA,
            pltpu.SemaphoreType.DMA,
            pltpu.SemaphoreType.DMA,
        ],
    )
    def sc_scatter_add(table_hbm, src_hbm, dst_hbm, zeros_hbm, out_hbm,
                       idx_s, di0, di1, rows0, rows1, acc,
                       sem0, sem1, semd0, semd1):
        c = lax.axis_index("c")
        s = lax.axis_index("s")
        w = c * NS + s
        base = w * EPW
        pltpu.sync_copy(src_hbm.at[pl.ds(base, EPW)], idx_s)
        pltpu.sync_copy(zeros_hbm.at[pl.ds(s * RPS, RPS)],
                        acc.at[pl.ds(s * RPS, RPS)])
        plsc.subcore_barrier()

        def wait_rows(buf, sem):
            pltpu.make_async_copy(table_hbm.at[idx_s.at[pl.ds(0, CH)]],
                                  buf, sem).wait()

        def wait_idx(buf, sem):
            pltpu.make_async_copy(dst_hbm.at[pl.ds(0, CH)], buf, sem).wait()

        pltpu.async_copy(dst_hbm.at[pl.ds(base, CH)], di0, semd0)
        pltpu.async_copy(table_hbm.at[idx_s.at[pl.ds(0, CH)]], rows0, sem0)

        def step(t, carry):
            g0 = t * 2
            pltpu.async_copy(dst_hbm.at[pl.ds(base + (g0 + 1) * CH, CH)],
                             di1, semd1)
            pltpu.async_copy(table_hbm.at[idx_s.at[pl.ds((g0 + 1) * CH, CH)]],
                             rows1, sem1)
            wait_rows(rows0, sem0)
            wait_idx(di0, semd0)
            pltpu.sync_copy(rows0, acc.at[di0], add=True)

            @pl.when(g0 + 2 < NCHUNK)
            def _():
                pltpu.async_copy(dst_hbm.at[pl.ds(base + (g0 + 2) * CH, CH)],
                                 di0, semd0)
                pltpu.async_copy(
                    table_hbm.at[idx_s.at[pl.ds((g0 + 2) * CH, CH)]], rows0, sem0)

            wait_rows(rows1, sem1)
            wait_idx(di1, semd1)
            pltpu.sync_copy(rows1, acc.at[di1], add=True)
            return carry

        lax.fori_loop(0, NCHUNK // 2, step, 0)
        plsc.subcore_barrier()
        pltpu.sync_copy(acc.at[pl.ds(s * RPS, RPS)],
                        out_hbm.at[c, pl.ds(s * RPS, RPS)])

    return sc_scatter_add


_sc_scatter_add_128 = _make_sc_scatter_add(128)

@functools.partial(
    pl.kernel,
    mesh=_sc_mesh,
    out_type=jax.ShapeDtypeStruct((E_PAD, 128), jnp.float32),
    scratch_types=[
        pltpu.VMEM((EPW,), jnp.int32),
        pltpu.VMEM((CH, 128), jnp.float32),
        pltpu.VMEM((CH, 128), jnp.float32),
        pltpu.VMEM((CH, 128), jnp.float32),
        pltpu.VMEM((CH, 128), jnp.float32),
        pltpu.SemaphoreType.DMA,
        pltpu.SemaphoreType.DMA,
        pltpu.SemaphoreType.DMA,
        pltpu.SemaphoreType.DMA,
    ],
)
def _sc_gather_edges(table_hbm, src_hbm, out_hbm, idx_s,
                     rows0, rows1, rows2, rows3, sem0, sem1, sem2, sem3):
    c = lax.axis_index("c")
    s = lax.axis_index("s")
    w = c * NS + s
    base = w * EPW
    pltpu.sync_copy(src_hbm.at[pl.ds(base, EPW)], idx_s)

    bufs = (rows0, rows1, rows2, rows3)
    sems = (sem0, sem1, sem2, sem3)

    def start(g, b):
        pltpu.async_copy(table_hbm.at[idx_s.at[pl.ds(g * CH, CH)]],
                         bufs[b], sems[b])

    def wait_rows(b):
        pltpu.make_async_copy(table_hbm.at[idx_s.at[pl.ds(0, CH)]],
                              bufs[b], sems[b]).wait()

    for b in range(4):
        start(b, b)

    def step(t, carry):
        g0 = t * 4
        for b in range(4):
            wait_rows(b)
            pltpu.sync_copy(bufs[b], out_hbm.at[pl.ds(base + (g0 + b) * CH, CH)])

            @pl.when(g0 + 4 + b < NCHUNK)
            def _():
                start(g0 + 4 + b, b)
        return carry

    lax.fori_loop(0, NCHUNK // 4, step, 0)


_GCH = 80


@functools.partial(
    pl.kernel,
    mesh=_sc_mesh,
    out_type=jax.ShapeDtypeStruct((K_PAD, 128), jnp.float32),
    scratch_types=[
        pltpu.VMEM((_GCH,), jnp.int32),
        pltpu.VMEM((_GCH, 128), jnp.float32),
        pltpu.SemaphoreType.DMA,
    ],
)
def _sc_gather_rows(table_hbm, perm_hbm, out_hbm, idx, rows, sem):
    c = lax.axis_index("c")
    s = lax.axis_index("s")
    w = c * NS + s

    def step(g, carry):
        base = w * (K_PAD // NW) + g * _GCH
        pltpu.sync_copy(perm_hbm.at[pl.ds(base, _GCH)], idx)
        pltpu.async_copy(table_hbm.at[idx], rows, sem).wait()
        pltpu.sync_copy(rows, out_hbm.at[pl.ds(base, _GCH)])
        return carry

    lax.fori_loop(0, K_PAD // NW // _GCH, step, 0)


_BR = 1024
_GRID = N_PAD // _BR


def _dinv2_block(mk, dp):
    deg = dp[0, :, 0:1] + dp[1, :, 0:1]
    return mk[:, 0:1] * lax.rsqrt(deg + 1.0)


def _tc_c_body(x1_ref, sc_ref, mk_ref, dp_ref, wb_ref, h2_ref, hsa_ref, hsb_ref):
    x1 = x1_ref[...]
    xp = mk_ref[:, 0:1] * sc_ref[:, 0:1] * x1
    h2 = jnp.dot(xp, wb_ref[...], preferred_element_type=jnp.float32)
    dinv2 = _dinv2_block(mk_ref[...], dp_ref[...])
    hs2 = dinv2 * h2
    h2_ref[...] = h2
    hsa_ref[...] = hs2[:, :128]
    hsb_ref[...] = hs2[:, 128:]


def _tc_d_body(a2a_ref, a2b_ref, h2_ref, x1_ref, mk_ref, dp_ref,
               wua_ref, wub_ref, bb_ref, h3_ref, hs3_ref):
    dinv2 = _dinv2_block(mk_ref[...], dp_ref[...])
    acc2a = a2a_ref[0] + a2a_ref[1]
    acc2b = a2b_ref[0] + a2b_ref[1]
    h2 = h2_ref[...]
    xb = jnp.concatenate(
        [dinv2 * acc2a + dinv2 * dinv2 * h2[:, :128],
         dinv2 * acc2b + dinv2 * dinv2 * h2[:, 128:]], axis=1) + bb_ref[...]
    h3 = (jnp.dot(xb, wua_ref[...], preferred_element_type=jnp.float32)
          + jnp.dot(x1_ref[...], wub_ref[...], preferred_element_type=jnp.float32))
    h3_ref[...] = h3
    hs3_ref[...] = dinv2 * h3


def _tc_e_body(a3_ref, h3_ref, mk_ref, dp_ref, wf_ref, bu_ref, h4_ref, hs4_ref):
    dinv2 = _dinv2_block(mk_ref[...], dp_ref[...])
    acc3 = a3_ref[0] + a3_ref[1]
    xu = dinv2 * acc3 + dinv2 * dinv2 * h3_ref[...] + bu_ref[...]
    h4 = jnp.dot(xu, wf_ref[...], preferred_element_type=jnp.float32)
    h4_ref[...] = h4
    hs4_ref[...] = dinv2 * h4


def _tc_f_body(a4_ref, h4_ref, mk_ref, dp_ref, bf_ref, out_ref):
    dinv2 = _dinv2_block(mk_ref[...], dp_ref[...])
    acc4 = a4_ref[0] + a4_ref[1]
    out_ref[...] = dinv2 * acc4 + dinv2 * dinv2 * h4_ref[...] + bf_ref[...]


def _rows_spec(width):
    return pl.BlockSpec((_BR, width), lambda i: (i, 0))


def _part_spec(width):
    return pl.BlockSpec((2, _BR, width), lambda i: (0, i, 0))


def _full_spec(shape):
    return pl.BlockSpec(shape, lambda i: tuple(0 for _ in shape))


def _tc_call(body, in_specs, out_specs, out_shapes):
    return pl.pallas_call(
        body,
        grid=(_GRID,),
        in_specs=in_specs,
        out_specs=out_specs,
        out_shape=out_shapes,
    )


def kernel(x, edge_index, edge_attr, W_down, b_down, p_pool,
           W_bottle, b_bottle, W_up, b_up, W_final, b_final):
    N = x.shape[0]
    src = edge_index[0]
    dst = edge_index[1]
    E = src.shape[0]

    pad_e = E_PAD - E
    src_p = jnp.concatenate([src, jnp.full((pad_e,), N_NODES, jnp.int32)])
    dst_p = jnp.concatenate([dst, jnp.full((pad_e,), N_NODES, jnp.int32)])

    h1 = x @ W_down
    deg1 = jnp.zeros((N,), jnp.float32).at[dst].add(jnp.ones((E,), jnp.float32)) + 1.0
    dinv1 = 1.0 / jnp.sqrt(deg1)
    norm1 = dinv1[src] * dinv1[dst]
    g1 = _sc_gather_edges(jnp.pad(h1, ((0, N_PAD - N), (0, 0))), src_p)[:E]
    x1 = jnp.zeros((N, 128), jnp.float32).at[dst].add(norm1[:, None] * g1)
    x1 = x1 + (dinv1 * dinv1)[:, None] * h1
    x1 = x1 + b_down
    score = jnp.tanh((x1 @ p_pool) / jnp.sqrt(jnp.sum(p_pool * p_pool)))
    k = (N + 1) // 2
    topscore, perm = jax.lax.top_k(score, k)
    mask = jnp.zeros((N,), jnp.float32).at[perm].set(1.0)
    perm_p = jnp.concatenate(
        [perm.astype(jnp.int32), jnp.full((K_PAD - k,), N_NODES, jnp.int32)])
    zeros128 = jnp.zeros((N_PAD, 128), jnp.float32)
    x1p = jnp.pad(x1, ((0, N_PAD - N), (0, 0)))
    mask_b = jnp.pad(jnp.broadcast_to(mask[:, None], (N, 128)),
                     ((0, N_PAD - N), (0, 0)))
    score_b = jnp.pad(jnp.broadcast_to(score[:, None], (N, 128)),
                      ((0, N_PAD - N), (0, 0)))
    bb = b_bottle.reshape(1, 256)
    bu = b_up.reshape(1, 128)
    bf = b_final.reshape(1, 128)
    wua = W_up[:256]
    wub = W_up[256:]

    deg_p = _sc_scatter_add_128(mask_b, src_p, dst_p, zeros128)

    h2, hs2a, hs2b = _tc_call(
        _tc_c_body,
        [_rows_spec(128), _rows_spec(128), _rows_spec(128), _part_spec(128),
         _full_spec((128, 256))],
        [_rows_spec(256), _rows_spec(128), _rows_spec(128)],
        [jax.ShapeDtypeStruct((N_PAD, 256), jnp.float32),
         jax.ShapeDtypeStruct((N_PAD, 128), jnp.float32),
         jax.ShapeDtypeStruct((N_PAD, 128), jnp.float32)],
    )(x1p, score_b, mask_b, deg_p, W_bottle)

    a2a = _sc_scatter_add_128(hs2a, src_p, dst_p, zeros128)
    a2b = _sc_scatter_add_128(hs2b, src_p, dst_p, zeros128)

    h3, hs3 = _tc_call(
        _tc_d_body,
        [_part_spec(128), _part_spec(128), _rows_spec(256), _rows_spec(128),
         _rows_spec(128), _part_spec(128), _full_spec((256, 128)),
         _full_spec((128, 128)), _full_spec((1, 256))],
        [_rows_spec(128), _rows_spec(128)],
        [jax.ShapeDtypeStruct((N_PAD, 128), jnp.float32),
         jax.ShapeDtypeStruct((N_PAD, 128), jnp.float32)],
    )(a2a, a2b, h2, x1p, mask_b, deg_p, wua, wub, bb)

    a3 = _sc_scatter_add_128(hs3, src_p, dst_p, zeros128)

    h4, hs4 = _tc_call(
        _tc_e_body,
        [_part_spec(128), _rows_spec(128), _rows_spec(128), _part_spec(128),
         _full_spec((128, 128)), _full_spec((1, 128))],
        [_rows_spec(128), _rows_spec(128)],
        [jax.ShapeDtypeStruct((N_PAD, 128), jnp.float32),
         jax.ShapeDtypeStruct((N_PAD, 128), jnp.float32)],
    )(a3, h3, mask_b, deg_p, W_final, bu)

    a4 = _sc_scatter_add_128(hs4, src_p, dst_p, zeros128)

    out_full = _tc_call(
        _tc_f_body,
        [_part_spec(128), _rows_spec(128), _rows_spec(128), _part_spec(128),
         _full_spec((1, 128))],
        _rows_spec(128),
        jax.ShapeDtypeStruct((N_PAD, 128), jnp.float32),
    )(a4, h4, mask_b, deg_p, bf)

    out_p = _sc_gather_rows(out_full, perm_p)
    return out_p[:k]
